# Initial kernel scaffold; baseline (speedup 1.0000x reference)
import functools

import jax
import jax.numpy as jnp
from jax import lax
from jax.experimental import pallas as pl
from jax.experimental.pallas import tpu as pltpu
from jax.experimental.pallas import tpu_sc as plsc

B = 128 * 16 * 16 * 16
D = 32
NC = 2
NS = 16
NW = NC * NS
BPW = B // NW
ROW = 128
NROWS = BPW // ROW
NB = 8
NSTEP = NROWS // NB

_mesh = plsc.VectorSubcoreMesh(core_axis_name="c", subcore_axis_name="s")


@functools.partial(
    pl.kernel,
    mesh=_mesh,
    out_type=jax.ShapeDtypeStruct((B, D), jnp.float32),
    scratch_types=[
        pltpu.VMEM((NROWS, ROW), jnp.int32),
        pltpu.VMEM((NB, ROW, D), jnp.float32),
        pltpu.SemaphoreType.DMA,
        pltpu.SemaphoreType.DMA,
    ],
)
def _gather(idx_hbm, table_hbm, out_hbm, idx_v, rbuf, gsem, osem):
    wid = lax.axis_index("s") * NC + lax.axis_index("c")
    base = wid * BPW
    pltpu.sync_copy(idx_hbm.at[wid], idx_v)

    def step(g, carry):
        gh = []
        for b in range(NB):
            j = g * NB + b
            gh.append(pltpu.async_copy(table_hbm.at[idx_v.at[j]], rbuf.at[b], gsem))
        oh = []
        for b in range(NB):
            gh[b].wait()
            j = g * NB + b
            oh.append(
                pltpu.async_copy(rbuf.at[b], out_hbm.at[pl.ds(base + j * ROW, ROW)], osem)
            )
        for b in range(NB):
            oh[b].wait()
        return carry

    lax.fori_loop(0, NSTEP, step, 0)


def kernel(inputs, table):
    idx = inputs.reshape(NW, NROWS, ROW)
    out = _gather(idx, table)
    b, x, y = inputs.shape[0], inputs.shape[1], inputs.shape[2]
    return out.reshape(b, x, y, -1)


# speedup vs baseline: 10.8077x; 10.8077x over previous
import functools

import jax
import jax.numpy as jnp
from jax import lax
from jax.experimental import pallas as pl
from jax.experimental.pallas import tpu as pltpu
from jax.experimental.pallas import tpu_sc as plsc

B = 128 * 16 * 16 * 16
D = 32
NC = 2
NS = 16
NW = NC * NS
BPW = B // NW
ROW = 128
NROWS = BPW // ROW
NB = 8
NSTEP = NROWS // NB

_mesh = plsc.VectorSubcoreMesh(core_axis_name="c", subcore_axis_name="s")


@functools.partial(
    pl.kernel,
    mesh=_mesh,
    compiler_params=pltpu.CompilerParams(use_tc_tiling_on_sc=False),
    out_type=jax.ShapeDtypeStruct((B, D), jnp.float32),
    scratch_types=[
        pltpu.VMEM((NROWS, ROW), jnp.int32),
        pltpu.VMEM((NB, ROW, D), jnp.float32),
        pltpu.SemaphoreType.DMA,
        pltpu.SemaphoreType.DMA,
    ],
)
def _gather(idx_hbm, table_hbm, out_hbm, idx_v, rbuf, gsem, osem):
    wid = lax.axis_index("s") * NC + lax.axis_index("c")
    base = wid * BPW
    pltpu.sync_copy(idx_hbm.at[wid], idx_v)

    def step(g, carry):
        gh = []
        for b in range(NB):
            j = g * NB + b
            gh.append(pltpu.async_copy(table_hbm.at[idx_v.at[j]], rbuf.at[b], gsem))
        oh = []
        for b in range(NB):
            gh[b].wait()
            j = g * NB + b
            oh.append(
                pltpu.async_copy(rbuf.at[b], out_hbm.at[pl.ds(base + j * ROW, ROW)], osem)
            )
        for b in range(NB):
            oh[b].wait()
        return carry

    lax.fori_loop(0, NSTEP, step, 0)


def kernel(inputs, table):
    idx = inputs.reshape(NW, NROWS, ROW)
    out = _gather(idx, table)
    b, x, y = inputs.shape[0], inputs.shape[1], inputs.shape[2]
    return out.reshape(b, x, y, -1)
